# Initial kernel scaffold; baseline (speedup 1.0000x reference)
import jax
import jax.numpy as jnp
from jax.experimental import pallas as pl


def kernel(x, edge_index, W, b):
    raise NotImplementedError("write your pallas kernel here")


# speedup vs baseline: 8.6636x; 8.6636x over previous
import functools

import jax
import jax.numpy as jnp
from jax import lax
from jax.experimental import pallas as pl
from jax.experimental.pallas import tpu as pltpu
from jax.experimental.pallas import tpu_sc as plsc

N = 10000
E = 320000
D = 128

NC = 2
NS = 16
NW = NC * NS

BATCH = 128
NB = -(-E // (NW * BATCH))
E_PAD = NW * NB * BATCH
PAD_ROWS = 240
N_PAD = N + PAD_ROWS
ZROWS = N_PAD // NS

MM_BLK = 1000


def _mm_body(x_ref, w_ref, o_ref):
    o_ref[...] = jnp.dot(x_ref[...], w_ref[...],
                         preferred_element_type=jnp.float32)


def _i32(v):
    return jnp.asarray(v, jnp.int32)


def _matmul(x, w):
    return pl.pallas_call(
        _mm_body,
        grid=(N // MM_BLK,),
        in_specs=[
            pl.BlockSpec((MM_BLK, D), lambda i: (i, _i32(0))),
            pl.BlockSpec((D, D), lambda i: (_i32(0), _i32(0))),
        ],
        out_specs=pl.BlockSpec((MM_BLK, D), lambda i: (i, _i32(0))),
        out_shape=jax.ShapeDtypeStruct((N, D), jnp.float32),
    )(x, w)


def _comb_body(p_ref, b_ref, o_ref):
    o_ref[...] = p_ref[0] + p_ref[1] + b_ref[...]


def _combine(parts, b2d):
    return pl.pallas_call(
        _comb_body,
        grid=(N // MM_BLK,),
        in_specs=[
            pl.BlockSpec((2, MM_BLK, D), lambda i: (_i32(0), i, _i32(0))),
            pl.BlockSpec((1, D), lambda i: (_i32(0), _i32(0))),
        ],
        out_specs=pl.BlockSpec((MM_BLK, D), lambda i: (i, _i32(0))),
        out_shape=jax.ShapeDtypeStruct((N, D), jnp.float32),
    )(parts, b2d)


@functools.partial(
    pl.kernel,
    mesh=plsc.VectorSubcoreMesh(core_axis_name="c", subcore_axis_name="s"),
    out_type=jax.ShapeDtypeStruct((NC, N_PAD, D), jnp.float32),
    scratch_types=[
        pltpu.VMEM((NB, BATCH), jnp.int32),
        pltpu.VMEM((NB, BATCH), jnp.int32),
        pltpu.VMEM((BATCH, D), jnp.float32),
        pltpu.VMEM_SHARED((N_PAD, D), jnp.float32),
        pltpu.SemaphoreType.DMA,
    ],
)
def _sc_scatter(support_hbm, src_hbm, dst_hbm, zeros_hbm, out_hbm,
                src_v, dst_v, rows_v, acc_sh, sem):
    c = lax.axis_index("c")
    s = lax.axis_index("s")
    wid = s * NC + c

    pltpu.sync_copy(src_hbm.at[wid], src_v)
    pltpu.sync_copy(dst_hbm.at[wid], dst_v)

    pltpu.sync_copy(zeros_hbm.at[pl.ds(s * ZROWS, ZROWS)],
                    acc_sh.at[pl.ds(s * ZROWS, ZROWS)])
    plsc.subcore_barrier()

    def step(j, carry):
        pltpu.async_copy(support_hbm.at[src_v.at[j]], rows_v, sem).wait()
        pltpu.sync_copy(rows_v, acc_sh.at[dst_v.at[j]], add=True)
        return carry

    lax.fori_loop(jnp.int32(0), jnp.int32(NB), step, jnp.int32(0))

    plsc.subcore_barrier()
    pltpu.sync_copy(acc_sh.at[pl.ds(s * ZROWS, ZROWS)],
                    out_hbm.at[c, pl.ds(s * ZROWS, ZROWS)])


def kernel(x, edge_index, W, b):
    src = edge_index[0].astype(jnp.int32)
    dst = edge_index[1].astype(jnp.int32)
    npad = E_PAD - E
    pad_i = jnp.arange(npad, dtype=jnp.int32)
    src_p = jnp.concatenate([src, (pad_i * 67) % N]).reshape(NW, NB, BATCH)
    dst_p = jnp.concatenate([dst, N + (pad_i % PAD_ROWS)]).reshape(NW, NB, BATCH)
    zeros = jnp.zeros((N_PAD, D), jnp.float32)

    support = _matmul(x, W)
    parts = _sc_scatter(support, src_p, dst_p, zeros)
    return _combine(parts, b.reshape(1, D))


# speedup vs baseline: 10.7864x; 1.2450x over previous
import functools

import jax
import jax.numpy as jnp
from jax import lax
from jax.experimental import pallas as pl
from jax.experimental.pallas import tpu as pltpu
from jax.experimental.pallas import tpu_sc as plsc

N = 10000
E = 320000
D = 128

NC = 2
NS = 16
NW = NC * NS

BATCH = 128
NB = -(-E // (NW * BATCH))
E_PAD = NW * NB * BATCH
PAD_ROWS = 240
N_PAD = N + PAD_ROWS
ZROWS = N_PAD // NS

MM_BLK = 1000


def _mm_body(x_ref, w_ref, o_ref):
    o_ref[...] = jnp.dot(x_ref[...], w_ref[...],
                         preferred_element_type=jnp.float32)


def _i32(v):
    return jnp.asarray(v, jnp.int32)


def _matmul(x, w):
    return pl.pallas_call(
        _mm_body,
        grid=(N // MM_BLK,),
        in_specs=[
            pl.BlockSpec((MM_BLK, D), lambda i: (i, _i32(0))),
            pl.BlockSpec((D, D), lambda i: (_i32(0), _i32(0))),
        ],
        out_specs=pl.BlockSpec((MM_BLK, D), lambda i: (i, _i32(0))),
        out_shape=jax.ShapeDtypeStruct((N, D), jnp.float32),
    )(x, w)


def _comb_body(p_ref, b_ref, o_ref):
    o_ref[...] = p_ref[0] + p_ref[1] + b_ref[...]


def _combine(parts, b2d):
    return pl.pallas_call(
        _comb_body,
        grid=(N // MM_BLK,),
        in_specs=[
            pl.BlockSpec((2, MM_BLK, D), lambda i: (_i32(0), i, _i32(0))),
            pl.BlockSpec((1, D), lambda i: (_i32(0), _i32(0))),
        ],
        out_specs=pl.BlockSpec((MM_BLK, D), lambda i: (i, _i32(0))),
        out_shape=jax.ShapeDtypeStruct((N, D), jnp.float32),
    )(parts, b2d)


@functools.partial(
    pl.kernel,
    mesh=plsc.VectorSubcoreMesh(core_axis_name="c", subcore_axis_name="s"),
    out_type=jax.ShapeDtypeStruct((NC, N_PAD, D), jnp.float32),
    scratch_types=[
        pltpu.VMEM((2, 2, BATCH), jnp.int32),
        pltpu.VMEM((2, BATCH, D), jnp.float32),
        pltpu.VMEM_SHARED((N_PAD, D), jnp.float32),
        pltpu.SemaphoreType.DMA,
        pltpu.SemaphoreType.DMA,
    ],
)
def _sc_scatter(support_hbm, idx_hbm, zeros_hbm, out_hbm,
                idx_v, rows_v, acc_sh, sem_g, sem_i):
    c = lax.axis_index("c")
    s = lax.axis_index("s")
    wid = s * NC + c
    i0 = jnp.int32(0)
    i1 = jnp.int32(1)

    pltpu.sync_copy(zeros_hbm.at[pl.ds(s * ZROWS, ZROWS)],
                    acc_sh.at[pl.ds(s * ZROWS, ZROWS)])

    pltpu.sync_copy(idx_hbm.at[wid, i0], idx_v.at[i0])
    plsc.subcore_barrier()
    pltpu.async_copy(support_hbm.at[idx_v.at[i0, i0]], rows_v.at[i0], sem_g)
    pltpu.async_copy(idx_hbm.at[wid, i1], idx_v.at[i1], sem_i)

    def step(j, carry):
        p = j % 2
        q = (j + 1) % 2
        pltpu.make_async_copy(support_hbm.at[idx_v.at[p, i0]],
                              rows_v.at[p], sem_g).wait()

        @pl.when(j + 1 < NB)
        def _():
            pltpu.make_async_copy(idx_hbm.at[wid, j + 1], idx_v.at[q],
                                  sem_i).wait()
            pltpu.async_copy(support_hbm.at[idx_v.at[q, i0]], rows_v.at[q],
                             sem_g)

        pltpu.sync_copy(rows_v.at[p], acc_sh.at[idx_v.at[p, i1]], add=True)

        @pl.when(j + 2 < NB)
        def _():
            pltpu.async_copy(idx_hbm.at[wid, j + 2], idx_v.at[p], sem_i)

        return carry

    lax.fori_loop(jnp.int32(0), jnp.int32(NB), step, jnp.int32(0))

    plsc.subcore_barrier()
    pltpu.sync_copy(acc_sh.at[pl.ds(s * ZROWS, ZROWS)],
                    out_hbm.at[c, pl.ds(s * ZROWS, ZROWS)])


def kernel(x, edge_index, W, b):
    src = edge_index[0].astype(jnp.int32)
    dst = edge_index[1].astype(jnp.int32)
    npad = E_PAD - E
    pad_i = jnp.arange(npad, dtype=jnp.int32)
    src_p = jnp.concatenate([src, (pad_i * 67) % N]).reshape(NW, NB, BATCH)
    dst_p = jnp.concatenate([dst, N + (pad_i % PAD_ROWS)]).reshape(NW, NB, BATCH)
    idx = jnp.stack([src_p, dst_p], axis=2)
    zeros = jnp.zeros((N_PAD, D), jnp.float32)

    support = _matmul(x, W)
    parts = _sc_scatter(support, idx, zeros)
    return _combine(parts, b.reshape(1, D))


# speedup vs baseline: 11.3816x; 1.0552x over previous
import functools

import jax
import jax.numpy as jnp
from jax import lax
from jax.experimental import pallas as pl
from jax.experimental.pallas import tpu as pltpu
from jax.experimental.pallas import tpu_sc as plsc

N = 10000
E = 320000
D = 128

NC = 2
NS = 16
NW = NC * NS

BATCH = 96
NBUF = 3
NB = -(-E // (NW * BATCH))
E_PAD = NW * NB * BATCH
PAD_ROWS = 112
N_PAD = N + PAD_ROWS
ZROWS = N_PAD // NS

MM_BLK = 1000


def _mm_body(x_ref, w_ref, o_ref):
    o_ref[...] = jnp.dot(x_ref[...], w_ref[...],
                         preferred_element_type=jnp.float32)


def _i32(v):
    return jnp.asarray(v, jnp.int32)


def _matmul(x, w):
    return pl.pallas_call(
        _mm_body,
        grid=(N // MM_BLK,),
        in_specs=[
            pl.BlockSpec((MM_BLK, D), lambda i: (i, _i32(0))),
            pl.BlockSpec((D, D), lambda i: (_i32(0), _i32(0))),
        ],
        out_specs=pl.BlockSpec((MM_BLK, D), lambda i: (i, _i32(0))),
        out_shape=jax.ShapeDtypeStruct((N, D), jnp.float32),
    )(x, w)


def _comb_body(p_ref, b_ref, o_ref):
    o_ref[...] = p_ref[0] + p_ref[1] + b_ref[...]


def _combine(parts, b2d):
    return pl.pallas_call(
        _comb_body,
        grid=(N // MM_BLK,),
        in_specs=[
            pl.BlockSpec((2, MM_BLK, D), lambda i: (_i32(0), i, _i32(0))),
            pl.BlockSpec((1, D), lambda i: (_i32(0), _i32(0))),
        ],
        out_specs=pl.BlockSpec((MM_BLK, D), lambda i: (i, _i32(0))),
        out_shape=jax.ShapeDtypeStruct((N, D), jnp.float32),
    )(parts, b2d)


@functools.partial(
    pl.kernel,
    mesh=plsc.VectorSubcoreMesh(core_axis_name="c", subcore_axis_name="s"),
    out_type=jax.ShapeDtypeStruct((NC, N_PAD, D), jnp.float32),
    scratch_types=[
        pltpu.VMEM((NBUF, 2, BATCH), jnp.int32),
        pltpu.VMEM((NBUF, BATCH, D), jnp.float32),
        pltpu.VMEM_SHARED((N_PAD, D), jnp.float32),
        pltpu.SemaphoreType.DMA,
        pltpu.SemaphoreType.DMA,
    ],
)
def _sc_scatter(support_hbm, idx_hbm, zeros_hbm, out_hbm,
                idx_v, rows_v, acc_sh, sem_g, sem_i):
    c = lax.axis_index("c")
    s = lax.axis_index("s")
    wid = s * NC + c
    i0 = jnp.int32(0)
    i1 = jnp.int32(1)
    i2 = jnp.int32(2)

    pltpu.sync_copy(zeros_hbm.at[pl.ds(s * ZROWS, ZROWS)],
                    acc_sh.at[pl.ds(s * ZROWS, ZROWS)])

    pltpu.sync_copy(idx_hbm.at[wid, i0], idx_v.at[i0])
    plsc.subcore_barrier()
    pltpu.async_copy(support_hbm.at[idx_v.at[i0, i0]], rows_v.at[i0], sem_g)
    pltpu.sync_copy(idx_hbm.at[wid, i1], idx_v.at[i1])
    pltpu.async_copy(support_hbm.at[idx_v.at[i1, i0]], rows_v.at[i1], sem_g)
    pltpu.async_copy(idx_hbm.at[wid, i2], idx_v.at[i2], sem_i)

    def step(j, carry):
        p = j % NBUF
        pltpu.make_async_copy(support_hbm.at[idx_v.at[p, i0]],
                              rows_v.at[p], sem_g).wait()
        pltpu.sync_copy(rows_v.at[p], acc_sh.at[idx_v.at[p, i1]], add=True)

        @pl.when(j + 2 < NB)
        def _():
            q = (j + 2) % NBUF
            pltpu.make_async_copy(idx_hbm.at[wid, j + 2], idx_v.at[q],
                                  sem_i).wait()
            pltpu.async_copy(support_hbm.at[idx_v.at[q, i0]], rows_v.at[q],
                             sem_g)

        @pl.when(j + 3 < NB)
        def _():
            pltpu.async_copy(idx_hbm.at[wid, j + 3], idx_v.at[p], sem_i)

        return carry

    lax.fori_loop(jnp.int32(0), jnp.int32(NB), step, jnp.int32(0))

    plsc.subcore_barrier()
    pltpu.sync_copy(acc_sh.at[pl.ds(s * ZROWS, ZROWS)],
                    out_hbm.at[c, pl.ds(s * ZROWS, ZROWS)])


def kernel(x, edge_index, W, b):
    src = edge_index[0].astype(jnp.int32)
    dst = edge_index[1].astype(jnp.int32)
    npad = E_PAD - E
    pad_i = jnp.arange(npad, dtype=jnp.int32)
    src_p = jnp.concatenate([src, (pad_i * 67) % N]).reshape(NW, NB, BATCH)
    dst_p = jnp.concatenate([dst, N + (pad_i % PAD_ROWS)]).reshape(NW, NB, BATCH)
    idx = jnp.stack([src_p, dst_p], axis=2)
    zeros = jnp.zeros((N_PAD, D), jnp.float32)

    support = _matmul(x, W)
    parts = _sc_scatter(support, idx, zeros)
    return _combine(parts, b.reshape(1, D))


# speedup vs baseline: 11.9115x; 1.0466x over previous
import functools

import jax
import jax.numpy as jnp
from jax import lax
from jax.experimental import pallas as pl
from jax.experimental.pallas import tpu as pltpu
from jax.experimental.pallas import tpu_sc as plsc

N = 10000
E = 320000
D = 128

NC = 2
NS = 16
NW = NC * NS

BATCH = 96
NBUF = 3
NB = -(-E // (NW * BATCH))
E_PAD = NW * NB * BATCH
PAD_ROWS = 112
N_PAD = N + PAD_ROWS
ZROWS = N_PAD // NS

MM_BLK = 1000


def _i32(v):
    return jnp.asarray(v, jnp.int32)


def _mmc_body(p_ref, w_ref, b_ref, o_ref):
    agg = p_ref[0] + p_ref[1]
    o_ref[...] = jnp.dot(agg, w_ref[...],
                         preferred_element_type=jnp.float32) + b_ref[...]


def _matmul_combine(parts, w, b2d):
    return pl.pallas_call(
        _mmc_body,
        grid=(N // MM_BLK,),
        in_specs=[
            pl.BlockSpec((2, MM_BLK, D), lambda i: (_i32(0), i, _i32(0))),
            pl.BlockSpec((D, D), lambda i: (_i32(0), _i32(0))),
            pl.BlockSpec((1, D), lambda i: (_i32(0), _i32(0))),
        ],
        out_specs=pl.BlockSpec((MM_BLK, D), lambda i: (i, _i32(0))),
        out_shape=jax.ShapeDtypeStruct((N, D), jnp.float32),
    )(parts, w, b2d)


@functools.partial(
    pl.kernel,
    mesh=plsc.VectorSubcoreMesh(core_axis_name="c", subcore_axis_name="s"),
    out_type=jax.ShapeDtypeStruct((NC, N_PAD, D), jnp.float32),
    scratch_types=[
        pltpu.VMEM((NBUF, 2, BATCH), jnp.int32),
        pltpu.VMEM((NBUF, BATCH, D), jnp.float32),
        pltpu.VMEM_SHARED((N_PAD, D), jnp.float32),
        pltpu.SemaphoreType.DMA,
        pltpu.SemaphoreType.DMA,
    ],
)
def _sc_scatter(support_hbm, idx_hbm, zeros_hbm, out_hbm,
                idx_v, rows_v, acc_sh, sem_g, sem_i):
    c = lax.axis_index("c")
    s = lax.axis_index("s")
    wid = s * NC + c
    i0 = jnp.int32(0)
    i1 = jnp.int32(1)
    i2 = jnp.int32(2)

    pltpu.sync_copy(zeros_hbm.at[pl.ds(s * ZROWS, ZROWS)],
                    acc_sh.at[pl.ds(s * ZROWS, ZROWS)])

    pltpu.sync_copy(idx_hbm.at[wid, i0], idx_v.at[i0])
    plsc.subcore_barrier()
    pltpu.async_copy(support_hbm.at[idx_v.at[i0, i0]], rows_v.at[i0], sem_g)
    pltpu.sync_copy(idx_hbm.at[wid, i1], idx_v.at[i1])
    pltpu.async_copy(support_hbm.at[idx_v.at[i1, i0]], rows_v.at[i1], sem_g)
    pltpu.async_copy(idx_hbm.at[wid, i2], idx_v.at[i2], sem_i)

    def step(j, carry):
        p = j % NBUF
        pltpu.make_async_copy(support_hbm.at[idx_v.at[p, i0]],
                              rows_v.at[p], sem_g).wait()
        pltpu.sync_copy(rows_v.at[p], acc_sh.at[idx_v.at[p, i1]], add=True)

        @pl.when(j + 2 < NB)
        def _():
            q = (j + 2) % NBUF
            pltpu.make_async_copy(idx_hbm.at[wid, j + 2], idx_v.at[q],
                                  sem_i).wait()
            pltpu.async_copy(support_hbm.at[idx_v.at[q, i0]], rows_v.at[q],
                             sem_g)

        @pl.when(j + 3 < NB)
        def _():
            pltpu.async_copy(idx_hbm.at[wid, j + 3], idx_v.at[p], sem_i)

        return carry

    lax.fori_loop(jnp.int32(0), jnp.int32(NB), step, jnp.int32(0))

    plsc.subcore_barrier()
    pltpu.sync_copy(acc_sh.at[pl.ds(s * ZROWS, ZROWS)],
                    out_hbm.at[c, pl.ds(s * ZROWS, ZROWS)])


def kernel(x, edge_index, W, b):
    src = edge_index[0].astype(jnp.int32)
    dst = edge_index[1].astype(jnp.int32)
    npad = E_PAD - E
    pad_i = jnp.arange(npad, dtype=jnp.int32)
    src_p = jnp.concatenate([src, (pad_i * 67) % N]).reshape(NW, NB, BATCH)
    dst_p = jnp.concatenate([dst, N + (pad_i % PAD_ROWS)]).reshape(NW, NB, BATCH)
    idx = jnp.stack([src_p, dst_p], axis=2)
    zeros = jnp.zeros((N_PAD, D), jnp.float32)

    parts = _sc_scatter(x, idx, zeros)
    return _matmul_combine(parts, W, b.reshape(1, D))


# speedup vs baseline: 12.0865x; 1.0147x over previous
import functools

import jax
import jax.numpy as jnp
from jax import lax
from jax.experimental import pallas as pl
from jax.experimental.pallas import tpu as pltpu
from jax.experimental.pallas import tpu_sc as plsc

N = 10000
E = 320000
D = 128

NC = 2
NS = 16
NW = NC * NS

BATCH = 80
NBUF = 3
NB = E // (NW * BATCH)
N_PAD = 10112
ZROWS = N_PAD // NS

MM_BLK = 1000


def _i32(v):
    return jnp.asarray(v, jnp.int32)


def _mmc_body(p_ref, w_ref, b_ref, o_ref):
    agg = p_ref[0] + p_ref[1]
    o_ref[...] = jnp.dot(agg, w_ref[...],
                         preferred_element_type=jnp.float32) + b_ref[...]


def _matmul_combine(parts, w, b2d):
    return pl.pallas_call(
        _mmc_body,
        grid=(N // MM_BLK,),
        in_specs=[
            pl.BlockSpec((2, MM_BLK, D), lambda i: (_i32(0), i, _i32(0))),
            pl.BlockSpec((D, D), lambda i: (_i32(0), _i32(0))),
            pl.BlockSpec((1, D), lambda i: (_i32(0), _i32(0))),
        ],
        out_specs=pl.BlockSpec((MM_BLK, D), lambda i: (i, _i32(0))),
        out_shape=jax.ShapeDtypeStruct((N, D), jnp.float32),
    )(parts, w, b2d)


@functools.partial(
    pl.kernel,
    mesh=plsc.VectorSubcoreMesh(core_axis_name="c", subcore_axis_name="s"),
    out_type=jax.ShapeDtypeStruct((NC, N_PAD, D), jnp.float32),
    scratch_types=[
        pltpu.VMEM((NBUF, BATCH), jnp.int32),
        pltpu.VMEM((NBUF, BATCH), jnp.int32),
        pltpu.VMEM((NBUF, BATCH, D), jnp.float32),
        pltpu.VMEM_SHARED((N_PAD, D), jnp.float32),
        pltpu.SemaphoreType.DMA,
        pltpu.SemaphoreType.DMA,
    ],
)
def _sc_scatter(support_hbm, src_hbm, dst_hbm, zeros_hbm, out_hbm,
                src_v, dst_v, rows_v, acc_sh, sem_g, sem_i):
    c = lax.axis_index("c")
    s = lax.axis_index("s")
    wid = s * NC + c
    i0 = jnp.int32(0)
    i1 = jnp.int32(1)
    i2 = jnp.int32(2)

    pltpu.sync_copy(zeros_hbm.at[pl.ds(s * ZROWS, ZROWS)],
                    acc_sh.at[pl.ds(s * ZROWS, ZROWS)])

    pltpu.sync_copy(src_hbm.at[wid, i0], src_v.at[i0])
    pltpu.sync_copy(dst_hbm.at[wid, i0], dst_v.at[i0])
    plsc.subcore_barrier()
    pltpu.async_copy(support_hbm.at[src_v.at[i0]], rows_v.at[i0], sem_g)
    pltpu.sync_copy(src_hbm.at[wid, i1], src_v.at[i1])
    pltpu.sync_copy(dst_hbm.at[wid, i1], dst_v.at[i1])
    pltpu.async_copy(support_hbm.at[src_v.at[i1]], rows_v.at[i1], sem_g)
    pltpu.async_copy(src_hbm.at[wid, i2], src_v.at[i2], sem_i)
    pltpu.async_copy(dst_hbm.at[wid, i2], dst_v.at[i2], sem_i)

    def step(j, carry):
        p = j % NBUF
        pltpu.make_async_copy(support_hbm.at[src_v.at[p]],
                              rows_v.at[p], sem_g).wait()
        pltpu.sync_copy(rows_v.at[p], acc_sh.at[dst_v.at[p]], add=True)

        @pl.when(j + 2 < NB)
        def _():
            q = (j + 2) % NBUF
            pltpu.make_async_copy(src_hbm.at[wid, j + 2], src_v.at[q],
                                  sem_i).wait()
            pltpu.make_async_copy(dst_hbm.at[wid, j + 2], dst_v.at[q],
                                  sem_i).wait()
            pltpu.async_copy(support_hbm.at[src_v.at[q]], rows_v.at[q],
                             sem_g)

        @pl.when(j + 3 < NB)
        def _():
            pltpu.async_copy(src_hbm.at[wid, j + 3], src_v.at[p], sem_i)
            pltpu.async_copy(dst_hbm.at[wid, j + 3], dst_v.at[p], sem_i)

        return carry

    lax.fori_loop(jnp.int32(0), jnp.int32(NB), step, jnp.int32(0))

    plsc.subcore_barrier()
    pltpu.sync_copy(acc_sh.at[pl.ds(s * ZROWS, ZROWS)],
                    out_hbm.at[c, pl.ds(s * ZROWS, ZROWS)])


def kernel(x, edge_index, W, b):
    src_p = edge_index[0].astype(jnp.int32).reshape(NW, NB, BATCH)
    dst_p = edge_index[1].astype(jnp.int32).reshape(NW, NB, BATCH)
    zeros = jnp.zeros((N_PAD, D), jnp.float32)

    parts = _sc_scatter(x, src_p, dst_p, zeros)
    return _matmul_combine(parts, W, b.reshape(1, D))


# speedup vs baseline: 13.3351x; 1.1033x over previous
import functools

import jax
import jax.numpy as jnp
from jax import lax
from jax.experimental import pallas as pl
from jax.experimental.pallas import tpu as pltpu
from jax.experimental.pallas import tpu_sc as plsc

N = 10000
E = 320000
D = 128

NC = 2
NS = 16
NW = NC * NS

BATCH = 80
NBUF = 3
NB = E // (NW * BATCH)
N_PAD = 10112
ZROWS = N_PAD // NS

MM_BLK = 1000


def _i32(v):
    return jnp.asarray(v, jnp.int32)


def _mmc_body(p_ref, w_ref, b_ref, o_ref):
    agg = p_ref[0] + p_ref[1]
    o_ref[...] = jnp.dot(agg, w_ref[...],
                         preferred_element_type=jnp.float32) + b_ref[...]


def _matmul_combine(parts, w, b2d):
    return pl.pallas_call(
        _mmc_body,
        grid=(N // MM_BLK,),
        in_specs=[
            pl.BlockSpec((2, MM_BLK, D), lambda i: (_i32(0), i, _i32(0))),
            pl.BlockSpec((D, D), lambda i: (_i32(0), _i32(0))),
            pl.BlockSpec((1, D), lambda i: (_i32(0), _i32(0))),
        ],
        out_specs=pl.BlockSpec((MM_BLK, D), lambda i: (i, _i32(0))),
        out_shape=jax.ShapeDtypeStruct((N, D), jnp.float32),
    )(parts, w, b2d)


@functools.partial(
    pl.kernel,
    mesh=plsc.VectorSubcoreMesh(core_axis_name="c", subcore_axis_name="s"),
    out_type=jax.ShapeDtypeStruct((NC, N_PAD, D), jnp.float32),
    scratch_types=[
        pltpu.VMEM((NBUF, BATCH), jnp.int32),
        pltpu.VMEM((NBUF, BATCH), jnp.int32),
        pltpu.VMEM((NBUF, BATCH, D), jnp.float32),
        pltpu.VMEM_SHARED((N_PAD, D), jnp.float32),
        pltpu.SemaphoreType.DMA,
        pltpu.SemaphoreType.DMA,
    ],
)
def _sc_scatter(support_hbm, ei_hbm, zeros_hbm, out_hbm,
                src_v, dst_v, rows_v, acc_sh, sem_g, sem_i):
    c = lax.axis_index("c")
    s = lax.axis_index("s")
    wid = s * NC + c
    base = wid * (NB * BATCH)
    i0 = jnp.int32(0)
    i1 = jnp.int32(1)
    i2 = jnp.int32(2)

    def src_at(j):
        return ei_hbm.at[pl.ds(base + j * BATCH, BATCH)]

    def dst_at(j):
        return ei_hbm.at[pl.ds(E + base + j * BATCH, BATCH)]

    pltpu.sync_copy(zeros_hbm, acc_sh.at[pl.ds(s * ZROWS, ZROWS)])

    pltpu.sync_copy(src_at(i0), src_v.at[i0])
    pltpu.sync_copy(dst_at(i0), dst_v.at[i0])
    plsc.subcore_barrier()
    pltpu.async_copy(support_hbm.at[src_v.at[i0]], rows_v.at[i0], sem_g)
    pltpu.sync_copy(src_at(i1), src_v.at[i1])
    pltpu.sync_copy(dst_at(i1), dst_v.at[i1])
    pltpu.async_copy(support_hbm.at[src_v.at[i1]], rows_v.at[i1], sem_g)
    pltpu.async_copy(src_at(i2), src_v.at[i2], sem_i)
    pltpu.async_copy(dst_at(i2), dst_v.at[i2], sem_i)

    def step(j, carry):
        p = j % NBUF
        pltpu.make_async_copy(support_hbm.at[src_v.at[p]],
                              rows_v.at[p], sem_g).wait()
        pltpu.sync_copy(rows_v.at[p], acc_sh.at[dst_v.at[p]], add=True)

        @pl.when(j + 2 < NB)
        def _():
            q = (j + 2) % NBUF
            pltpu.make_async_copy(src_at(j + 2), src_v.at[q], sem_i).wait()
            pltpu.make_async_copy(dst_at(j + 2), dst_v.at[q], sem_i).wait()
            pltpu.async_copy(support_hbm.at[src_v.at[q]], rows_v.at[q],
                             sem_g)

        @pl.when(j + 3 < NB)
        def _():
            pltpu.async_copy(src_at(j + 3), src_v.at[p], sem_i)
            pltpu.async_copy(dst_at(j + 3), dst_v.at[p], sem_i)

        return carry

    lax.fori_loop(jnp.int32(0), jnp.int32(NB), step, jnp.int32(0))

    plsc.subcore_barrier()
    pltpu.sync_copy(acc_sh.at[pl.ds(s * ZROWS, ZROWS)],
                    out_hbm.at[c, pl.ds(s * ZROWS, ZROWS)])


def kernel(x, edge_index, W, b):
    ei32 = edge_index.astype(jnp.int32).reshape(-1)
    zeros = jnp.zeros((ZROWS, D), jnp.float32)

    parts = _sc_scatter(x, ei32, zeros)
    return _matmul_combine(parts, W, b.reshape(1, D))


# speedup vs baseline: 13.3738x; 1.0029x over previous
import functools

import jax
import jax.numpy as jnp
from jax import lax
from jax.experimental import pallas as pl
from jax.experimental.pallas import tpu as pltpu
from jax.experimental.pallas import tpu_sc as plsc

N = 10000
E = 320000
D = 128

NC = 2
NS = 16
NW = NC * NS

BATCH = 80
NBUF = 3
NB = E // (NW * BATCH)
N_PAD = 10112
ZROWS = N_PAD // NS

MM_BLK = 1000


def _i32(v):
    return jnp.asarray(v, jnp.int32)


def _mmc_body(p_ref, w_ref, b_ref, o_ref):
    agg = p_ref[0] + p_ref[1]
    o_ref[...] = jnp.dot(agg, w_ref[...],
                         preferred_element_type=jnp.float32) + b_ref[...]


def _matmul_combine(parts, w, b2d):
    return pl.pallas_call(
        _mmc_body,
        grid=(N // MM_BLK,),
        in_specs=[
            pl.BlockSpec((2, MM_BLK, D), lambda i: (_i32(0), i, _i32(0))),
            pl.BlockSpec((D, D), lambda i: (_i32(0), _i32(0))),
            pl.BlockSpec((1, D), lambda i: (_i32(0), _i32(0))),
        ],
        out_specs=pl.BlockSpec((MM_BLK, D), lambda i: (i, _i32(0))),
        out_shape=jax.ShapeDtypeStruct((N, D), jnp.float32),
    )(parts, w, b2d)


@functools.partial(
    pl.kernel,
    mesh=plsc.VectorSubcoreMesh(core_axis_name="c", subcore_axis_name="s"),
    out_type=jax.ShapeDtypeStruct((NC, N_PAD, D), jnp.float32),
    scratch_types=[
        pltpu.VMEM((NBUF, BATCH), jnp.int32),
        pltpu.VMEM((NBUF, BATCH), jnp.int32),
        pltpu.VMEM((NBUF, BATCH, D), jnp.float32),
        pltpu.VMEM_SHARED((N_PAD, D), jnp.float32),
        pltpu.SemaphoreType.DMA,
        pltpu.SemaphoreType.DMA,
    ],
)
def _sc_scatter(support_hbm, ei_hbm, zeros_hbm, out_hbm,
                src_v, dst_v, rows_v, acc_sh, sem_g, sem_i):
    c = lax.axis_index("c")
    s = lax.axis_index("s")
    wid = s * NC + c
    base = wid * (NB * BATCH)
    i0 = jnp.int32(0)
    i1 = jnp.int32(1)
    i2 = jnp.int32(2)

    def src_at(j):
        return ei_hbm.at[pl.ds(base + j * BATCH, BATCH)]

    def dst_at(j):
        return ei_hbm.at[pl.ds(E + base + j * BATCH, BATCH)]

    pltpu.sync_copy(src_at(i0), src_v.at[i0])
    pltpu.async_copy(support_hbm.at[src_v.at[i0]], rows_v.at[i0], sem_g)
    pltpu.sync_copy(src_at(i1), src_v.at[i1])
    pltpu.async_copy(support_hbm.at[src_v.at[i1]], rows_v.at[i1], sem_g)
    pltpu.sync_copy(dst_at(i0), dst_v.at[i0])
    pltpu.sync_copy(dst_at(i1), dst_v.at[i1])
    pltpu.async_copy(src_at(i2), src_v.at[i2], sem_i)
    pltpu.async_copy(dst_at(i2), dst_v.at[i2], sem_i)

    pltpu.sync_copy(zeros_hbm, acc_sh.at[pl.ds(s * ZROWS, ZROWS)])
    plsc.subcore_barrier()

    def step(j, carry):
        p = j % NBUF
        pltpu.make_async_copy(support_hbm.at[src_v.at[p]],
                              rows_v.at[p], sem_g).wait()
        pltpu.sync_copy(rows_v.at[p], acc_sh.at[dst_v.at[p]], add=True)

        @pl.when(j + 2 < NB)
        def _():
            q = (j + 2) % NBUF
            pltpu.make_async_copy(src_at(j + 2), src_v.at[q], sem_i).wait()
            pltpu.make_async_copy(dst_at(j + 2), dst_v.at[q], sem_i).wait()
            pltpu.async_copy(support_hbm.at[src_v.at[q]], rows_v.at[q],
                             sem_g)

        @pl.when(j + 3 < NB)
        def _():
            pltpu.async_copy(src_at(j + 3), src_v.at[p], sem_i)
            pltpu.async_copy(dst_at(j + 3), dst_v.at[p], sem_i)

        return carry

    lax.fori_loop(jnp.int32(0), jnp.int32(NB), step, jnp.int32(0))

    plsc.subcore_barrier()
    pltpu.sync_copy(acc_sh.at[pl.ds(s * ZROWS, ZROWS)],
                    out_hbm.at[c, pl.ds(s * ZROWS, ZROWS)])


def kernel(x, edge_index, W, b):
    ei32 = edge_index.astype(jnp.int32).reshape(-1)
    zeros = jnp.zeros((ZROWS, D), jnp.float32)

    parts = _sc_scatter(x, ei32, zeros)
    return _matmul_combine(parts, W, b.reshape(1, D))


# speedup vs baseline: 15.5635x; 1.1637x over previous
import functools

import jax
import jax.numpy as jnp
from jax import lax
from jax.experimental import pallas as pl
from jax.experimental.pallas import tpu as pltpu
from jax.experimental.pallas import tpu_sc as plsc

N = 10000
E = 320000
D = 128

NC = 2
NS = 16
NW = NC * NS

BATCH = 80
NBUF = 4
NB = E // (NW * BATCH)
N_PAD = 10112
ZROWS = N_PAD // NS

MM_BLK = 1000


def _i32(v):
    return jnp.asarray(v, jnp.int32)


def _mmc_body(p_ref, w_ref, b_ref, o_ref):
    agg = p_ref[0] + p_ref[1]
    o_ref[...] = jnp.dot(agg, w_ref[...],
                         preferred_element_type=jnp.float32) + b_ref[...]


def _matmul_combine(parts, w, b2d):
    return pl.pallas_call(
        _mmc_body,
        grid=(N // MM_BLK,),
        in_specs=[
            pl.BlockSpec((2, MM_BLK, D), lambda i: (_i32(0), i, _i32(0))),
            pl.BlockSpec((D, D), lambda i: (_i32(0), _i32(0))),
            pl.BlockSpec((1, D), lambda i: (_i32(0), _i32(0))),
        ],
        out_specs=pl.BlockSpec((MM_BLK, D), lambda i: (i, _i32(0))),
        out_shape=jax.ShapeDtypeStruct((N, D), jnp.float32),
    )(parts, w, b2d)


@functools.partial(
    pl.kernel,
    mesh=plsc.VectorSubcoreMesh(core_axis_name="c", subcore_axis_name="s"),
    out_type=jax.ShapeDtypeStruct((NC, N_PAD, D), jnp.float32),
    scratch_types=[
        pltpu.VMEM((NBUF, BATCH), jnp.int32),
        pltpu.VMEM((NBUF, BATCH), jnp.int32),
        pltpu.VMEM((NBUF, BATCH, D), jnp.float32),
        pltpu.VMEM_SHARED((N_PAD, D), jnp.float32),
        pltpu.SemaphoreType.DMA,
        pltpu.SemaphoreType.DMA,
    ],
)
def _sc_scatter(support_hbm, ei_hbm, zeros_hbm, out_hbm,
                src_v, dst_v, rows_v, acc_sh, sem_g, sem_i):
    c = lax.axis_index("c")
    s = lax.axis_index("s")
    wid = s * NC + c
    base = wid * (NB * BATCH)
    i0 = jnp.int32(0)
    i1 = jnp.int32(1)
    i2 = jnp.int32(2)

    def src_at(j):
        return ei_hbm.at[pl.ds(base + j * BATCH, BATCH)]

    def dst_at(j):
        return ei_hbm.at[pl.ds(E + base + j * BATCH, BATCH)]

    for k in range(NBUF - 1):
        ik = jnp.int32(k)
        pltpu.sync_copy(src_at(ik), src_v.at[ik])
        pltpu.async_copy(support_hbm.at[src_v.at[ik]], rows_v.at[ik], sem_g)
    for k in range(NBUF - 1):
        ik = jnp.int32(k)
        pltpu.sync_copy(dst_at(ik), dst_v.at[ik])
    ilast = jnp.int32(NBUF - 1)
    pltpu.async_copy(src_at(ilast), src_v.at[ilast], sem_i)
    pltpu.async_copy(dst_at(ilast), dst_v.at[ilast], sem_i)

    pltpu.sync_copy(zeros_hbm, acc_sh.at[pl.ds(s * ZROWS, ZROWS)])
    plsc.subcore_barrier()

    def step(j, carry):
        p = j % NBUF
        pltpu.make_async_copy(support_hbm.at[src_v.at[p]],
                              rows_v.at[p], sem_g).wait()
        pltpu.sync_copy(rows_v.at[p], acc_sh.at[dst_v.at[p]], add=True)

        @pl.when(j + NBUF - 1 < NB)
        def _():
            q = (j + NBUF - 1) % NBUF
            pltpu.make_async_copy(src_at(j + NBUF - 1), src_v.at[q],
                                  sem_i).wait()
            pltpu.make_async_copy(dst_at(j + NBUF - 1), dst_v.at[q],
                                  sem_i).wait()
            pltpu.async_copy(support_hbm.at[src_v.at[q]], rows_v.at[q],
                             sem_g)

        @pl.when(j + NBUF < NB)
        def _():
            pltpu.async_copy(src_at(j + NBUF), src_v.at[p], sem_i)
            pltpu.async_copy(dst_at(j + NBUF), dst_v.at[p], sem_i)

        return carry

    lax.fori_loop(jnp.int32(0), jnp.int32(NB), step, jnp.int32(0))

    plsc.subcore_barrier()
    pltpu.sync_copy(acc_sh.at[pl.ds(s * ZROWS, ZROWS)],
                    out_hbm.at[c, pl.ds(s * ZROWS, ZROWS)])


def kernel(x, edge_index, W, b):
    ei32 = edge_index.astype(jnp.int32).reshape(-1)
    zeros = jnp.zeros((ZROWS, D), jnp.float32)

    parts = _sc_scatter(x, ei32, zeros)
    return _matmul_combine(parts, W, b.reshape(1, D))
